# baseline (speedup 1.0000x reference)
import functools

import jax
import jax.numpy as jnp
from jax import lax
from jax.experimental import pallas as pl
from jax.experimental.pallas import tpu as pltpu
from jax.experimental.pallas import tpu_sc as plsc

HIDDEN = 768
L = 16
NVEC = HIDDEN // L
NC, NS = 2, 16
NW = NC * NS
B = 4
SEQ = 2048
TOKENS = B * SEQ
PPW = SEQ // NW
C = 16
PB = PPW // C
NCHUNK = B * PB
NBUF = 2
RB = 16
EPS = 1e-10


_GATHER_DNUMS = lax.GatherDimensionNumbers(
    offset_dims=(), collapsed_slice_dims=(0,), start_index_map=(0,))


def _shuffle(x, perm):
    return lax.gather(x, perm, _GATHER_DNUMS, slice_sizes=(1,),
                      mode=lax.GatherScatterMode.PROMISE_IN_BOUNDS)


def _bcast_sum(x):
    lanes = lax.iota(jnp.int32, L)
    for k in (1, 2, 4, 8):
        perm = lax.reshape(lanes ^ k, (L, 1))
        x = x + _shuffle(x, perm)
    return x


def _rsqrt_vec(v):
    i = lax.bitcast_convert_type(v, jnp.int32)
    i = jnp.int32(0x5F3759DF) - lax.shift_right_logical(i, 1)
    y = lax.bitcast_convert_type(i, jnp.float32)
    for _ in range(3):
        y = y * (1.5 - 0.5 * v * y * y)
    return y


_mesh = plsc.VectorSubcoreMesh(core_axis_name="c", subcore_axis_name="s")


@functools.partial(
    pl.kernel,
    out_type=jax.ShapeDtypeStruct((TOKENS, HIDDEN), jnp.float32),
    mesh=_mesh,
    scratch_types=[
        pltpu.VMEM((B * PPW,), jnp.int32),
        pltpu.VMEM((NBUF * C, HIDDEN), jnp.float32),
        pltpu.VMEM((PPW, HIDDEN), jnp.float32),
        pltpu.VMEM((NBUF * C, HIDDEN), jnp.float32),
        pltpu.VMEM((HIDDEN,), jnp.float32),
        pltpu.VMEM((HIDDEN,), jnp.float32),
        pltpu.VMEM((HIDDEN,), jnp.float32),
        pltpu.SemaphoreType.DMA,
        pltpu.SemaphoreType.DMA,
        pltpu.SemaphoreType.DMA,
        pltpu.SemaphoreType.DMA,
        pltpu.SemaphoreType.DMA,
        pltpu.SemaphoreType.DMA,
        pltpu.SemaphoreType.DMA,
        pltpu.SemaphoreType.DMA,
    ],
)
def _emb_ln_kernel(ids_hbm, word_hbm, pos_hbm, type_hbm, gamma_hbm, beta_hbm,
                   out_hbm, idx_v, wbuf, pbuf, obuf, t_v, g_v, b_v,
                   gsem0, gsem1, gsem2, psem, isem, osem0, osem1, osem2):
    gsems = (gsem0, gsem1, gsem2)
    osems = (osem0, osem1, osem2)
    wid = lax.axis_index("s") * NC + lax.axis_index("c")
    s0 = wid * PPW

    pltpu.async_copy(pos_hbm.at[pl.ds(s0, PPW)], pbuf, psem)
    for b in range(B):
        pltpu.async_copy(ids_hbm.at[pl.ds(b * SEQ + s0, PPW)],
                         idx_v.at[pl.ds(b * PPW, PPW)], isem)
    pltpu.sync_copy(type_hbm, t_v)
    pltpu.sync_copy(gamma_hbm, g_v)
    pltpu.sync_copy(beta_hbm, b_v)
    pltpu.make_async_copy(ids_hbm.at[pl.ds(0, B * PPW)], idx_v, isem).wait()

    def start_gather(g, gsem):
        boff = lax.rem(g, NBUF) * C
        pltpu.async_copy(word_hbm.at[idx_v.at[pl.ds(g * C, C)]],
                         wbuf.at[pl.ds(boff, C)], gsem)

    for p in range(NBUF):
        start_gather(p, gsems[p])

    def wait_rows(n, dst, sem):
        pltpu.make_async_copy(pos_hbm.at[pl.ds(0, n)], dst, sem).wait()

    pltpu.make_async_copy(pos_hbm.at[pl.ds(0, PPW)], pbuf, psem).wait()

    def compute_block(boff, prow):
        zeros = tuple(jnp.zeros((L,), jnp.float32) for _ in range(2 * RB))

        @plsc.parallel_loop(0, HIDDEN, L, unroll=4, carry=zeros)
        def acc_body(o, c):
            sl = pl.ds(o, L)
            tv = t_v[sl]
            ws = [wbuf[boff + r, sl] for r in range(RB)]
            ps = [pbuf[prow + r, sl] for r in range(RB)]
            xs = [ws[r] + ps[r] + tv for r in range(RB)]
            for r in range(RB):
                wbuf[boff + r, sl] = xs[r]
            return (tuple(c[r] + xs[r] for r in range(RB)) +
                    tuple(c[RB + r] + xs[r] * xs[r] for r in range(RB)))

        accs = acc_body
        mean = [None] * RB
        inv = [None] * RB
        for r in range(RB):
            mean[r] = _bcast_sum(accs[r]) * (1.0 / HIDDEN)
            var = _bcast_sum(accs[RB + r]) * (1.0 / HIDDEN) - mean[r] * mean[r]
            inv[r] = _rsqrt_vec(var + EPS)

        @plsc.parallel_loop(0, HIDDEN, L, unroll=4)
        def norm_body(o):
            sl = pl.ds(o, L)
            gv = g_v[sl]
            bv = b_v[sl]
            xs = [wbuf[boff + r, sl] for r in range(RB)]
            ys = [(xs[r] - mean[r]) * inv[r] * gv + bv for r in range(RB)]
            for r in range(RB):
                obuf[boff + r, sl] = ys[r]

    def chunk_body(g, carry):
        slot = lax.rem(g, NBUF)
        boff = slot * C
        b = lax.div(g, PB)
        prow0 = (g - b * PB) * C
        out0 = b * SEQ + s0 + prow0

        for p in range(NBUF):
            @pl.when(slot == p)
            def _(p=p):
                wait_rows(C, wbuf.at[pl.ds(p * C, C)], gsems[p])

                @pl.when(g >= NBUF)
                def _():
                    wait_rows(C, obuf.at[pl.ds(p * C, C)], osems[p])

        def block_body(blk, carry2):
            compute_block(boff + blk * RB, prow0 + blk * RB)
            return carry2

        lax.fori_loop(0, C // RB, block_body, jnp.int32(0))

        for p in range(NBUF):
            @pl.when(slot == p)
            def _(p=p):
                pltpu.async_copy(obuf.at[pl.ds(p * C, C)],
                                 out_hbm.at[pl.ds(out0, C)], osems[p])

                @pl.when(g + NBUF < NCHUNK)
                def _():
                    start_gather(g + NBUF, gsems[p])

        return carry

    lax.fori_loop(0, NCHUNK, chunk_body, jnp.int32(0))

    for p in range(NBUF):
        wait_rows(C, obuf.at[pl.ds(p * C, C)], osems[p])


def kernel(input_ids, word_emb, pos_emb, type_emb, gamma, beta):
    ids = input_ids.reshape(-1).astype(jnp.int32)
    out = _emb_ln_kernel(ids, word_emb, pos_emb[:SEQ], type_emb[0],
                         gamma, beta)
    b, s = input_ids.shape
    return out.reshape(b, s, HIDDEN)


# speedup vs baseline: 1.0755x; 1.0755x over previous
import functools

import jax
import jax.numpy as jnp
from jax import lax
from jax.experimental import pallas as pl
from jax.experimental.pallas import tpu as pltpu
from jax.experimental.pallas import tpu_sc as plsc

HIDDEN = 768
L = 16
NVEC = HIDDEN // L
NC, NS = 2, 16
NW = NC * NS
B = 4
SEQ = 2048
TOKENS = B * SEQ
PPW = SEQ // NW
C = 16
PB = PPW // C
NCHUNK = B * PB
NBUF = 2
RB = 8
EPS = 1e-10


_GATHER_DNUMS = lax.GatherDimensionNumbers(
    offset_dims=(), collapsed_slice_dims=(0,), start_index_map=(0,))


def _shuffle(x, perm):
    return lax.gather(x, perm, _GATHER_DNUMS, slice_sizes=(1,),
                      mode=lax.GatherScatterMode.PROMISE_IN_BOUNDS)


def _bcast_sum(x):
    lanes = lax.iota(jnp.int32, L)
    for k in (1, 2, 4, 8):
        perm = lax.reshape(lanes ^ k, (L, 1))
        x = x + _shuffle(x, perm)
    return x


def _rsqrt_vec(v):
    i = lax.bitcast_convert_type(v, jnp.int32)
    i = jnp.int32(0x5F3759DF) - lax.shift_right_logical(i, 1)
    y = lax.bitcast_convert_type(i, jnp.float32)
    for _ in range(3):
        y = y * (1.5 - 0.5 * v * y * y)
    return y


_mesh = plsc.VectorSubcoreMesh(core_axis_name="c", subcore_axis_name="s")


@functools.partial(
    pl.kernel,
    out_type=jax.ShapeDtypeStruct((TOKENS, HIDDEN), jnp.float32),
    mesh=_mesh,
    scratch_types=[
        pltpu.VMEM((B * PPW,), jnp.int32),
        pltpu.VMEM((NBUF * C, HIDDEN), jnp.float32),
        pltpu.VMEM((PPW, HIDDEN), jnp.float32),
        pltpu.VMEM((NBUF * C, HIDDEN), jnp.float32),
        pltpu.VMEM((HIDDEN,), jnp.float32),
        pltpu.VMEM((HIDDEN,), jnp.float32),
        pltpu.VMEM((HIDDEN,), jnp.float32),
        pltpu.SemaphoreType.DMA,
        pltpu.SemaphoreType.DMA,
        pltpu.SemaphoreType.DMA,
        pltpu.SemaphoreType.DMA,
        pltpu.SemaphoreType.DMA,
        pltpu.SemaphoreType.DMA,
        pltpu.SemaphoreType.DMA,
        pltpu.SemaphoreType.DMA,
    ],
)
def _emb_ln_kernel(ids_hbm, word_hbm, pos_hbm, type_hbm, gamma_hbm, beta_hbm,
                   out_hbm, idx_v, wbuf, pbuf, obuf, t_v, g_v, b_v,
                   gsem0, gsem1, gsem2, psem, isem, osem0, osem1, osem2):
    gsems = (gsem0, gsem1, gsem2)
    osems = (osem0, osem1, osem2)
    wid = lax.axis_index("s") * NC + lax.axis_index("c")
    s0 = wid * PPW

    pltpu.async_copy(pos_hbm.at[pl.ds(s0, PPW)], pbuf, psem)
    for b in range(B):
        pltpu.async_copy(ids_hbm.at[pl.ds(b * SEQ + s0, PPW)],
                         idx_v.at[pl.ds(b * PPW, PPW)], isem)
    pltpu.sync_copy(type_hbm, t_v)
    pltpu.sync_copy(gamma_hbm, g_v)
    pltpu.sync_copy(beta_hbm, b_v)
    pltpu.make_async_copy(ids_hbm.at[pl.ds(0, B * PPW)], idx_v, isem).wait()

    def start_gather(g, gsem):
        boff = lax.rem(g, NBUF) * C
        pltpu.async_copy(word_hbm.at[idx_v.at[pl.ds(g * C, C)]],
                         wbuf.at[pl.ds(boff, C)], gsem)

    for p in range(NBUF):
        start_gather(p, gsems[p])

    def wait_rows(n, dst, sem):
        pltpu.make_async_copy(pos_hbm.at[pl.ds(0, n)], dst, sem).wait()

    pltpu.make_async_copy(pos_hbm.at[pl.ds(0, PPW)], pbuf, psem).wait()

    def compute_block(boff, prow):
        zeros = tuple(jnp.zeros((L,), jnp.float32) for _ in range(2 * RB))

        @plsc.parallel_loop(0, HIDDEN, L, unroll=4, carry=zeros)
        def acc_body(o, c):
            sl = pl.ds(o, L)
            tv = t_v[sl]
            ws = [wbuf[boff + r, sl] for r in range(RB)]
            ps = [pbuf[prow + r, sl] for r in range(RB)]
            xs = [ws[r] + ps[r] + tv for r in range(RB)]
            for r in range(RB):
                wbuf[boff + r, sl] = xs[r]
            return (tuple(c[r] + xs[r] for r in range(RB)) +
                    tuple(c[RB + r] + xs[r] * xs[r] for r in range(RB)))

        accs = acc_body
        mean = [None] * RB
        inv = [None] * RB
        for r in range(RB):
            mean[r] = _bcast_sum(accs[r]) * (1.0 / HIDDEN)
            var = _bcast_sum(accs[RB + r]) * (1.0 / HIDDEN) - mean[r] * mean[r]
            inv[r] = _rsqrt_vec(var + EPS)

        @plsc.parallel_loop(0, HIDDEN, L, unroll=4)
        def norm_body(o):
            sl = pl.ds(o, L)
            gv = g_v[sl]
            bv = b_v[sl]
            xs = [wbuf[boff + r, sl] for r in range(RB)]
            ys = [(xs[r] - mean[r]) * inv[r] * gv + bv for r in range(RB)]
            for r in range(RB):
                obuf[boff + r, sl] = ys[r]

    def chunk_body(g, carry):
        slot = lax.rem(g, NBUF)
        boff = slot * C
        b = lax.div(g, PB)
        prow0 = (g - b * PB) * C
        out0 = b * SEQ + s0 + prow0

        for p in range(NBUF):
            @pl.when(slot == p)
            def _(p=p):
                wait_rows(C, wbuf.at[pl.ds(p * C, C)], gsems[p])

                @pl.when(g >= NBUF)
                def _():
                    wait_rows(C, obuf.at[pl.ds(p * C, C)], osems[p])

        def block_body(blk, carry2):
            compute_block(boff + blk * RB, prow0 + blk * RB)
            return carry2

        lax.fori_loop(0, C // RB, block_body, jnp.int32(0))

        for p in range(NBUF):
            @pl.when(slot == p)
            def _(p=p):
                pltpu.async_copy(obuf.at[pl.ds(p * C, C)],
                                 out_hbm.at[pl.ds(out0, C)], osems[p])

                @pl.when(g + NBUF < NCHUNK)
                def _():
                    start_gather(g + NBUF, gsems[p])

        return carry

    lax.fori_loop(0, NCHUNK, chunk_body, jnp.int32(0))

    for p in range(NBUF):
        wait_rows(C, obuf.at[pl.ds(p * C, C)], osems[p])


def kernel(input_ids, word_emb, pos_emb, type_emb, gamma, beta):
    ids = input_ids.reshape(-1).astype(jnp.int32)
    out = _emb_ln_kernel(ids, word_emb, pos_emb[:SEQ], type_emb[0],
                         gamma, beta)
    b, s = input_ids.shape
    return out.reshape(b, s, HIDDEN)
